# baseline (speedup 1.0000x reference)
import functools
import math

import jax
import jax.numpy as jnp
from jax import lax
from jax.experimental import pallas as pl
from jax.experimental.pallas import tpu as pltpu
from jax.experimental.pallas import tpu_sc as plsc

EMBED_DIM = 64
SCALE = math.sqrt(EMBED_DIM)

NUM_CORES = 2
NUM_SUBCORES = 16
NUM_WORKERS = NUM_CORES * NUM_SUBCORES
LANES = 16

KB = 4
ROWS_PER_ITER = 4


def _make_kernel(batch: int, hist: int):
    assert batch % NUM_WORKERS == 0
    b_per_w = batch // NUM_WORKERS
    assert b_per_w % KB == 0
    n_chunks = b_per_w // KB
    rows_per_chunk = KB * hist

    mesh = plsc.VectorSubcoreMesh(
        core_axis_name="c", subcore_axis_name="s"
    )

    @functools.partial(
        pl.kernel,
        mesh=mesh,
        compiler_params=pltpu.CompilerParams(use_tc_tiling_on_sc=False),
        out_type=jax.ShapeDtypeStruct((batch, hist, EMBED_DIM), jnp.float32),
        scratch_types=[
            pltpu.VMEM((KB, hist), jnp.int32),
            pltpu.VMEM((KB, hist), jnp.int32),
            pltpu.VMEM((KB, hist, EMBED_DIM), jnp.float32),
            pltpu.VMEM((KB, hist, EMBED_DIM), jnp.float32),
            pltpu.SemaphoreType.DMA,
            pltpu.SemaphoreType.DMA,
            pltpu.SemaphoreType.DMA,
            pltpu.SemaphoreType.DMA,
        ],
    )
    def emb_kernel(x_hbm, table_hbm, out_hbm, idx0, idx1, rows0, rows1,
                   sg0, sg1, ss0, ss1):
        wid = lax.axis_index("s") * NUM_CORES + lax.axis_index("c")
        base = wid * b_per_w

        idx_v = (idx0, idx1)
        rows_v = (rows0, rows1)
        sg = (sg0, sg1)
        ss = (ss0, ss1)

        def start_gathers(chunk_i, b):
            bo = base + chunk_i * KB
            pltpu.sync_copy(x_hbm.at[pl.ds(bo, KB)], idx_v[b])
            handles = []
            for j in range(KB):
                handles.append(
                    pltpu.async_copy(
                        table_hbm.at[idx_v[b].at[j]], rows_v[b].at[j], sg[b]
                    )
                )
            return handles

        def scale_rows(b):
            rv = rows_v[b]
            for j in range(KB):
                rj = rv.at[j]

                def body(r0, c, rj=rj):
                    r = r0 * ROWS_PER_ITER
                    for dr in range(ROWS_PER_ITER):
                        for q in range(EMBED_DIM // LANES):
                            sl = rj[r + dr, pl.ds(q * LANES, LANES)]
                            rj[r + dr, pl.ds(q * LANES, LANES)] = sl * SCALE
                    return c

                lax.fori_loop(0, hist // ROWS_PER_ITER, body, 0)

        gather_h = [None, None]
        store_h = [None, None]

        gather_h[0] = start_gathers(0, 0)
        for i in range(n_chunks):
            b = i % 2
            if i + 1 < n_chunks:
                nb = (i + 1) % 2
                if store_h[nb] is not None:
                    store_h[nb].wait()
                    store_h[nb] = None
                gather_h[nb] = start_gathers(i + 1, nb)
            for h in gather_h[b]:
                h.wait()
            scale_rows(b)
            bo = base + i * KB
            store_h[b] = pltpu.async_copy(
                rows_v[b], out_hbm.at[pl.ds(bo, KB)], ss[b]
            )
        for b in range(2):
            if store_h[b] is not None:
                store_h[b].wait()

    return emb_kernel


def kernel(x, table):
    b, h = x.shape
    return _make_kernel(b, h)(x, table)


# speedup vs baseline: 1.2504x; 1.2504x over previous
import functools
import math

import jax
import jax.numpy as jnp
from jax import lax
from jax.experimental import pallas as pl
from jax.experimental.pallas import tpu as pltpu
from jax.experimental.pallas import tpu_sc as plsc
from jax.experimental.layout import Format, Layout, with_layout_constraint

EMBED_DIM = 64
SCALE = math.sqrt(EMBED_DIM)

NUM_CORES = 2
NUM_SUBCORES = 16
NUM_WORKERS = NUM_CORES * NUM_SUBCORES
LANES = 16

KB = 4
ROWS_PER_ITER = 4


def _make_kernel(batch: int, hist: int):
    assert batch % NUM_WORKERS == 0
    b_per_w = batch // NUM_WORKERS
    assert b_per_w % KB == 0
    n_chunks = b_per_w // KB
    rows_per_chunk = KB * hist

    mesh = plsc.VectorSubcoreMesh(
        core_axis_name="c", subcore_axis_name="s"
    )

    @functools.partial(
        pl.kernel,
        mesh=mesh,
        compiler_params=pltpu.CompilerParams(use_tc_tiling_on_sc=False),
        out_type=jax.ShapeDtypeStruct((batch, hist, EMBED_DIM), jnp.float32),
        scratch_types=[
            pltpu.VMEM((KB, hist), jnp.int32),
            pltpu.VMEM((KB, hist), jnp.int32),
            pltpu.VMEM((KB, hist, EMBED_DIM), jnp.float32),
            pltpu.VMEM((KB, hist, EMBED_DIM), jnp.float32),
            pltpu.SemaphoreType.DMA,
            pltpu.SemaphoreType.DMA,
            pltpu.SemaphoreType.DMA,
            pltpu.SemaphoreType.DMA,
        ],
    )
    def emb_kernel(x_hbm, table_hbm, out_hbm, idx0, idx1, rows0, rows1,
                   sg0, sg1, ss0, ss1):
        wid = lax.axis_index("s") * NUM_CORES + lax.axis_index("c")
        base = wid * b_per_w

        idx_v = (idx0, idx1)
        rows_v = (rows0, rows1)
        sg = (sg0, sg1)
        ss = (ss0, ss1)

        def start_gathers(chunk_i, b):
            bo = base + chunk_i * KB
            pltpu.sync_copy(x_hbm.at[pl.ds(bo, KB)], idx_v[b])
            handles = []
            for j in range(KB):
                handles.append(
                    pltpu.async_copy(
                        table_hbm.at[idx_v[b].at[j]], rows_v[b].at[j], sg[b]
                    )
                )
            return handles

        def scale_rows(b):
            rv = rows_v[b]
            for j in range(KB):
                rj = rv.at[j]

                def body(r0, c, rj=rj):
                    r = r0 * ROWS_PER_ITER
                    for dr in range(ROWS_PER_ITER):
                        for q in range(EMBED_DIM // LANES):
                            sl = rj[r + dr, pl.ds(q * LANES, LANES)]
                            rj[r + dr, pl.ds(q * LANES, LANES)] = sl * SCALE
                    return c

                lax.fori_loop(0, hist // ROWS_PER_ITER, body, 0)

        gather_h = [None, None]
        store_h = [None, None]

        gather_h[0] = start_gathers(0, 0)
        for i in range(n_chunks):
            b = i % 2
            if i + 1 < n_chunks:
                nb = (i + 1) % 2
                if store_h[nb] is not None:
                    store_h[nb].wait()
                    store_h[nb] = None
                gather_h[nb] = start_gathers(i + 1, nb)
            for h in gather_h[b]:
                h.wait()
            scale_rows(b)
            bo = base + i * KB
            store_h[b] = pltpu.async_copy(
                rows_v[b], out_hbm.at[pl.ds(bo, KB)], ss[b]
            )
        for b in range(2):
            if store_h[b] is not None:
                store_h[b].wait()

    return emb_kernel


def kernel(x, table):
    b, h = x.shape
    table = with_layout_constraint(
        table, Layout(major_to_minor=(0, 1))
    )
    return _make_kernel(b, h)(x, table)


# speedup vs baseline: 1.2511x; 1.0005x over previous
import functools
import math

import jax
import jax.numpy as jnp
from jax import lax
from jax.experimental import pallas as pl
from jax.experimental.pallas import tpu as pltpu
from jax.experimental.pallas import tpu_sc as plsc
from jax.experimental.layout import Format, Layout, with_layout_constraint

EMBED_DIM = 64
SCALE = math.sqrt(EMBED_DIM)

NUM_CORES = 2
NUM_SUBCORES = 16
NUM_WORKERS = NUM_CORES * NUM_SUBCORES
LANES = 16

KB = 4
ROWS_PER_ITER = 4


def _make_kernel(batch: int, hist: int):
    assert batch % NUM_WORKERS == 0
    b_per_w = batch // NUM_WORKERS
    assert b_per_w % KB == 0
    n_chunks = b_per_w // KB
    rows_per_chunk = KB * hist

    mesh = plsc.VectorSubcoreMesh(
        core_axis_name="c", subcore_axis_name="s"
    )

    @functools.partial(
        pl.kernel,
        mesh=mesh,
        compiler_params=pltpu.CompilerParams(use_tc_tiling_on_sc=False),
        out_type=jax.ShapeDtypeStruct((batch, hist, EMBED_DIM), jnp.float32),
        scratch_types=[
            pltpu.VMEM((KB, hist), jnp.int32),
            pltpu.VMEM((KB, hist), jnp.int32),
            pltpu.VMEM((KB, hist, EMBED_DIM), jnp.float32),
            pltpu.VMEM((KB, hist, EMBED_DIM), jnp.float32),
            pltpu.SemaphoreType.DMA,
            pltpu.SemaphoreType.DMA,
            pltpu.SemaphoreType.DMA,
            pltpu.SemaphoreType.DMA,
        ],
    )
    def emb_kernel(x_hbm, table_hbm, out_hbm, idx0, idx1, rows0, rows1,
                   sg0, sg1, ss0, ss1):
        wid = lax.axis_index("s") * NUM_CORES + lax.axis_index("c")
        base = wid * b_per_w

        idx_v = (idx0, idx1)
        rows_v = (rows0, rows1)
        sg = (sg0, sg1)
        ss = (ss0, ss1)

        def start_gathers(chunk_i, b):
            bo = base + chunk_i * KB
            pltpu.sync_copy(x_hbm.at[pl.ds(bo, KB)], idx_v[b])
            handles = []
            for j in range(KB):
                handles.append(
                    pltpu.async_copy(
                        table_hbm.at[idx_v[b].at[j]], rows_v[b].at[j], sg[b]
                    )
                )
            return handles

        def scale_rows(b):
            rv = rows_v[b]
            for j in range(KB):
                rj = rv.at[j]

                def body(r0, c, rj=rj):
                    r = r0 * ROWS_PER_ITER
                    for dr in range(ROWS_PER_ITER):
                        for q in range(EMBED_DIM // LANES):
                            sl = rj[r + dr, pl.ds(q * LANES, LANES)]
                            rj[r + dr, pl.ds(q * LANES, LANES)] = sl * SCALE
                    return c

                lax.fori_loop(0, hist // ROWS_PER_ITER, body, 0)

        gather_h = [None, None]
        store_h = [None, None]

        gather_h[0] = start_gathers(0, 0)
        for i in range(n_chunks):
            b = i % 2
            if i + 1 < n_chunks:
                nb = (i + 1) % 2
                if store_h[nb] is not None:
                    store_h[nb].wait()
                    store_h[nb] = None
                gather_h[nb] = start_gathers(i + 1, nb)
            for h in gather_h[b]:
                h.wait()
            scale_rows(b)
            bo = base + i * KB
            store_h[b] = pltpu.async_copy(
                rows_v[b], out_hbm.at[pl.ds(bo, KB)], ss[b]
            )
        for b in range(2):
            if store_h[b] is not None:
                store_h[b].wait()

    return emb_kernel


def kernel(x, table):
    b, h = x.shape
    table = with_layout_constraint(
        jax.lax.optimization_barrier(table), Layout(major_to_minor=(0, 1))
    )
    out = _make_kernel(b, h)(x, table)
    out = with_layout_constraint(
        out, Layout(major_to_minor=(1, 2, 0), tiling=((8, 128),))
    )
    return out
